# baseline (speedup 1.0000x reference)
import functools

import jax
import jax.numpy as jnp
from jax import lax
from jax.experimental import pallas as pl
from jax.experimental.pallas import tpu as pltpu
from jax.experimental.pallas import tpu_sc as plsc

N = 10000
E = 320000
D = 128

NC = 2
NS = 16
NW = NC * NS
EPW = E // NW
K = 40
CH = EPW // K
NB = 5
BCH = CH // NB
BE = BCH * K
RB = 4
RPT = 624
TAIL = N - NS * RPT

_mesh = plsc.VectorSubcoreMesh(
    core_axis_name="c", subcore_axis_name="s", num_cores=NC, num_subcores=NS
)


def _scale_rows(rows, rbase, w_v, wbase):
    for g0, ge in ((0, 16), (16, 16), (32, 8)):
        w16 = w_v[pl.ds(wbase + g0, 16)]
        for e in range(ge):
            s16 = w16.at[jnp.full((16,), e, jnp.int32)].get(
                mode="promise_in_bounds")
            i = rbase + g0 + e
            for d in range(D // 16):
                sl = pl.ds(d * 16, 16)
                rows[i, sl] = rows[i, sl] * s16


def _sc_body(x_hbm, src_hbm, dst_hbm, w_hbm, zeros_hbm, out_hbm,
             acc_sh, src_v, dst_v, w_v, ring, sem_g, sem_s):
    cid = lax.axis_index("c")
    sid = lax.axis_index("s")
    wid = cid * NS + sid

    def _gather(cc):
        sidx = cc % RB
        soff = pl.multiple_of(sidx * K, 8)
        off = pl.multiple_of(cc * K, 8)
        pltpu.async_copy(x_hbm.at[src_v.at[pl.ds(off, K)]],
                         ring.at[pl.ds(soff, K)], sem_g.at[sidx])

    def _drain(sem, sidx):
        pltpu.make_async_copy(x_hbm.at[pl.ds(0, K)],
                              ring.at[pl.ds(0, K)], sem.at[sidx]).wait()

    pltpu.sync_copy(zeros_hbm, acc_sh.at[pl.ds(sid * RPT, RPT)])

    @pl.when(sid == 0)
    def _zero_tail():
        pltpu.sync_copy(zeros_hbm.at[pl.ds(0, TAIL)],
                        acc_sh.at[pl.ds(NS * RPT, TAIL)])
    plsc.subcore_barrier()

    def block(bk, carry0):
        base = wid * EPW + bk * BE
        pltpu.sync_copy(src_hbm.at[pl.ds(base, BE)], src_v)
        pltpu.sync_copy(w_hbm.at[pl.ds(base, BE)], w_v.at[pl.ds(0, BE)])
        pltpu.sync_copy(dst_hbm.at[wid * NB + bk], dst_v)

        for c0 in range(RB - 1):
            _gather(c0)

        def chunk(c, carry):
            sidx = c % RB
            soff = pl.multiple_of(sidx * K, 8)
            _drain(sem_g, sidx)
            _scale_rows(ring, soff, w_v, c * K)
            pltpu.async_copy(ring.at[pl.ds(soff, K)],
                             acc_sh.at[dst_v.at[c]], sem_s.at[sidx],
                             add=True)

            @pl.when(c < BCH - RB + 1)
            def _():
                @pl.when(c > 0)
                def _():
                    _drain(sem_s, (c - 1) % RB)
                _gather(c + RB - 1)
            return carry

        lax.fori_loop(0, BCH, chunk, 0)
        for j in range(RB):
            _drain(sem_s, (BCH - RB + j) % RB)
        return carry0

    lax.fori_loop(0, NB, block, 0)

    plsc.subcore_barrier()
    pltpu.sync_copy(acc_sh.at[pl.ds(sid * RPT, RPT)],
                    out_hbm.at[cid, pl.ds(sid * RPT, RPT)])

    @pl.when(sid == 0)
    def _copy_tail():
        pltpu.sync_copy(acc_sh.at[pl.ds(NS * RPT, TAIL)],
                        out_hbm.at[cid, pl.ds(NS * RPT, TAIL)])


_sc_scatter = functools.partial(
    pl.kernel,
    out_type=jax.ShapeDtypeStruct((NC, N, D), jnp.float32),
    mesh=_mesh,
    scratch_types=[
        pltpu.VMEM_SHARED((N, D), jnp.float32),
        pltpu.VMEM((BE,), jnp.int32),
        pltpu.VMEM((BCH, K), jnp.int32),
        pltpu.VMEM((BE + 16,), jnp.float32),
        pltpu.VMEM((RB * K, D), jnp.float32),
        pltpu.SemaphoreType.DMA((RB,)),
        pltpu.SemaphoreType.DMA((RB,)),
    ],
)(_sc_body)


_BN = 2000


def _tc_body(p_ref, w_ref, b_ref, o_ref):
    acc = p_ref[0] + p_ref[1]
    o_ref[...] = (
        jnp.dot(acc, w_ref[...], preferred_element_type=jnp.float32) + b_ref[...]
    )


def _tc_finish(partials, W, b):
    return pl.pallas_call(
        _tc_body,
        grid=(N // _BN,),
        in_specs=[
            pl.BlockSpec((NC, _BN, D), lambda i: (0, i, 0)),
            pl.BlockSpec((D, D), lambda i: (0, 0)),
            pl.BlockSpec((1, D), lambda i: (0, 0)),
        ],
        out_specs=pl.BlockSpec((_BN, D), lambda i: (i, 0)),
        out_shape=jax.ShapeDtypeStruct((N, D), jnp.float32),
    )(partials, W, b.reshape(1, D))


def kernel(X, edge_index, edge_weight, W, b):
    src = edge_index[0]
    dst = edge_index[1].reshape(NW * NB, BCH, K)
    ew = edge_weight
    zeros = jnp.zeros((RPT, D), jnp.float32)
    partials = _sc_scatter(X, src, dst, ew, zeros)
    return _tc_finish(partials, W, b)


# speedup vs baseline: 1.0203x; 1.0203x over previous
import functools

import jax
import jax.numpy as jnp
from jax import lax
from jax.experimental import pallas as pl
from jax.experimental.pallas import tpu as pltpu
from jax.experimental.pallas import tpu_sc as plsc

N = 10000
E = 320000
D = 128

NC = 2
NS = 16
NW = NC * NS
EPW = E // NW
K = 40
CH = EPW // K
NB = 5
BCH = CH // NB
BE = BCH * K
RB = 5
RPT = 624
TAIL = N - NS * RPT

_mesh = plsc.VectorSubcoreMesh(
    core_axis_name="c", subcore_axis_name="s", num_cores=NC, num_subcores=NS
)


def _scale_rows(rows, rbase, w_v, wbase):
    for g0, ge in ((0, 16), (16, 16), (32, 8)):
        w16 = w_v[pl.ds(wbase + g0, 16)]
        for e in range(ge):
            s16 = w16.at[jnp.full((16,), e, jnp.int32)].get(
                mode="promise_in_bounds")
            i = rbase + g0 + e
            for d in range(D // 16):
                sl = pl.ds(d * 16, 16)
                rows[i, sl] = rows[i, sl] * s16


def _sc_body(x_hbm, src_hbm, dst_hbm, w_hbm, zeros_hbm, out_hbm,
             acc_sh, src_v, dst_v, w_v, ring, sem_g, sem_s):
    cid = lax.axis_index("c")
    sid = lax.axis_index("s")
    wid = cid * NS + sid

    def _gather(cc):
        sidx = cc % RB
        soff = pl.multiple_of(sidx * K, 8)
        off = pl.multiple_of(cc * K, 8)
        pltpu.async_copy(x_hbm.at[src_v.at[pl.ds(off, K)]],
                         ring.at[pl.ds(soff, K)], sem_g.at[sidx])

    def _drain(sem, sidx):
        pltpu.make_async_copy(x_hbm.at[pl.ds(0, K)],
                              ring.at[pl.ds(0, K)], sem.at[sidx]).wait()

    pltpu.sync_copy(zeros_hbm, acc_sh.at[pl.ds(sid * RPT, RPT)])

    @pl.when(sid == 0)
    def _zero_tail():
        pltpu.sync_copy(zeros_hbm.at[pl.ds(0, TAIL)],
                        acc_sh.at[pl.ds(NS * RPT, TAIL)])
    plsc.subcore_barrier()

    def block(bk, carry0):
        base = wid * EPW + bk * BE
        pltpu.sync_copy(src_hbm.at[pl.ds(base, BE)], src_v)
        pltpu.sync_copy(w_hbm.at[pl.ds(base, BE)], w_v.at[pl.ds(0, BE)])
        pltpu.sync_copy(dst_hbm.at[wid * NB + bk], dst_v)

        for c0 in range(RB - 1):
            _gather(c0)

        def chunk(c, carry):
            sidx = c % RB
            soff = pl.multiple_of(sidx * K, 8)
            _drain(sem_g, sidx)
            _scale_rows(ring, soff, w_v, c * K)
            pltpu.async_copy(ring.at[pl.ds(soff, K)],
                             acc_sh.at[dst_v.at[c]], sem_s.at[sidx],
                             add=True)

            @pl.when(c < BCH - RB + 1)
            def _():
                @pl.when(c > 0)
                def _():
                    _drain(sem_s, (c - 1) % RB)
                _gather(c + RB - 1)
            return carry

        lax.fori_loop(0, BCH, chunk, 0)
        for j in range(RB):
            _drain(sem_s, (BCH - RB + j) % RB)
        return carry0

    lax.fori_loop(0, NB, block, 0)

    plsc.subcore_barrier()
    pltpu.sync_copy(acc_sh.at[pl.ds(sid * RPT, RPT)],
                    out_hbm.at[cid, pl.ds(sid * RPT, RPT)])

    @pl.when(sid == 0)
    def _copy_tail():
        pltpu.sync_copy(acc_sh.at[pl.ds(NS * RPT, TAIL)],
                        out_hbm.at[cid, pl.ds(NS * RPT, TAIL)])


_sc_scatter = functools.partial(
    pl.kernel,
    out_type=jax.ShapeDtypeStruct((NC, N, D), jnp.float32),
    mesh=_mesh,
    scratch_types=[
        pltpu.VMEM_SHARED((N, D), jnp.float32),
        pltpu.VMEM((BE,), jnp.int32),
        pltpu.VMEM((BCH, K), jnp.int32),
        pltpu.VMEM((BE + 16,), jnp.float32),
        pltpu.VMEM((RB * K, D), jnp.float32),
        pltpu.SemaphoreType.DMA((RB,)),
        pltpu.SemaphoreType.DMA((RB,)),
    ],
)(_sc_body)


_BN = 2000


def _tc_body(p_ref, w_ref, b_ref, o_ref):
    acc = p_ref[0] + p_ref[1]
    o_ref[...] = (
        jnp.dot(acc, w_ref[...], preferred_element_type=jnp.float32) + b_ref[...]
    )


def _tc_finish(partials, W, b):
    return pl.pallas_call(
        _tc_body,
        grid=(N // _BN,),
        in_specs=[
            pl.BlockSpec((NC, _BN, D), lambda i: (0, i, 0)),
            pl.BlockSpec((D, D), lambda i: (0, 0)),
            pl.BlockSpec((1, D), lambda i: (0, 0)),
        ],
        out_specs=pl.BlockSpec((_BN, D), lambda i: (i, 0)),
        out_shape=jax.ShapeDtypeStruct((N, D), jnp.float32),
    )(partials, W, b.reshape(1, D))


def kernel(X, edge_index, edge_weight, W, b):
    src = edge_index[0]
    dst = edge_index[1].reshape(NW * NB, BCH, K)
    ew = edge_weight
    zeros = jnp.zeros((RPT, D), jnp.float32)
    partials = _sc_scatter(X, src, dst, ew, zeros)
    return _tc_finish(partials, W, b)
